# Initial kernel scaffold; baseline (speedup 1.0000x reference)
import jax
import jax.numpy as jnp
from jax.experimental import pallas as pl


def kernel(out1, out2, anchor1, anchor2):
    raise NotImplementedError("write your pallas kernel here")


# speedup vs baseline: 4.5273x; 4.5273x over previous
import jax
import jax.numpy as jnp
from jax.experimental import pallas as pl
from jax.experimental.pallas import tpu as pltpu

_K = 64
_GAMMA = 1.0
_N_NODES = 10000
_D_FEAT = 256
_N_ANCHORS = 1024

_BA = 8
_N_PAD = 10240
_NB = 2048
_NI = (2 * _N_ANCHORS) // _BA
_NJ = _N_PAD // _NB


def _gather_body(a1_ref, a2_ref, r1_ref, r2_ref, o1_ref, o2_ref):
    o1_ref[...] = r1_ref[...]
    o2_ref[...] = r2_ref[...]


def _main_body(ae_ref, aep_ref, cand_ref, out_ref, dist_ref):
    i = pl.program_id(0)
    j = pl.program_id(1)

    cand = cand_ref[...]
    rows = []
    for a in range(_BA):
        av = ae_ref[a, :]
        rows.append(jnp.sum(jnp.abs(cand - av), axis=1))
    blk = jnp.stack(rows, axis=0)
    dist_ref[:, pl.ds(j * _NB, _NB)] = blk

    @pl.when(jnp.logical_and(i == 0, j == 0))
    def _():
        out_ref[...] = jnp.zeros((1, 1), jnp.float32)

    @pl.when(j == _NJ - 1)
    def _():
        d = dist_ref[...]
        bits = jax.lax.bitcast_convert_type(d, jnp.int32)
        drow = jnp.sum(jnp.abs(ae_ref[...] - aep_ref[...]), axis=1,
                       keepdims=True) + _GAMMA

        def step(it, carry):
            prefix, kk = carry
            b = 30 - it
            sh = jnp.right_shift(bits, b)
            cand0 = prefix * 2
            c0 = jnp.sum((sh == cand0).astype(jnp.int32), axis=1,
                         keepdims=True)
            take0 = kk <= c0
            prefix = jnp.where(take0, cand0, cand0 + 1)
            kk = jnp.where(take0, kk, kk - c0)
            return prefix, kk

        prefix0 = jnp.zeros((_BA, 1), jnp.int32)
        kk0 = jnp.full((_BA, 1), _K, jnp.int32)
        tbits, _ = jax.lax.fori_loop(0, 31, step, (prefix0, kk0))
        t = jax.lax.bitcast_convert_type(tbits, jnp.float32)
        less = d < t
        c = jnp.sum(less.astype(jnp.int32), axis=1, keepdims=True)
        contrib = jnp.where(less, jnp.maximum(drow - d, 0.0), 0.0)
        row_loss = (jnp.sum(contrib, axis=1, keepdims=True)
                    + (_K - c).astype(jnp.float32)
                    * jnp.maximum(drow - t, 0.0))
        out_ref[...] += jnp.sum(row_loss, keepdims=True).reshape(1, 1)


def _gather(anchor1, anchor2, out1, out2):
    grid_spec = pltpu.PrefetchScalarGridSpec(
        num_scalar_prefetch=2,
        grid=(_N_ANCHORS,),
        in_specs=[
            pl.BlockSpec((1, 1, _D_FEAT), lambda i, a1, a2: (a1[i], 0, 0)),
            pl.BlockSpec((1, 1, _D_FEAT), lambda i, a1, a2: (a2[i], 0, 0)),
        ],
        out_specs=[
            pl.BlockSpec((1, 1, _D_FEAT), lambda i, a1, a2: (i, 0, 0)),
            pl.BlockSpec((1, 1, _D_FEAT), lambda i, a1, a2: (i, 0, 0)),
        ],
    )
    o1, o2 = pl.pallas_call(
        _gather_body,
        grid_spec=grid_spec,
        out_shape=[
            jax.ShapeDtypeStruct((_N_ANCHORS, 1, _D_FEAT), jnp.float32),
            jax.ShapeDtypeStruct((_N_ANCHORS, 1, _D_FEAT), jnp.float32),
        ],
    )(anchor1, anchor2,
      out1.reshape(out1.shape[0], 1, _D_FEAT),
      out2.reshape(out2.shape[0], 1, _D_FEAT))
    return (o1.reshape(_N_ANCHORS, _D_FEAT), o2.reshape(_N_ANCHORS, _D_FEAT))


def _main(ae, cand_cat):
    half_i = _N_ANCHORS // _BA
    return pl.pallas_call(
        _main_body,
        grid=(_NI, _NJ),
        in_specs=[
            pl.BlockSpec((_BA, _D_FEAT), lambda i, j: (i, 0)),
            pl.BlockSpec((_BA, _D_FEAT), lambda i, j: ((i + half_i) % _NI, 0)),
            pl.BlockSpec((_NB, _D_FEAT),
                         lambda i, j: ((i // half_i) * _NJ + j, 0)),
        ],
        out_specs=pl.BlockSpec((1, 1), lambda i, j: (0, 0)),
        out_shape=jax.ShapeDtypeStruct((1, 1), jnp.float32),
        scratch_shapes=[pltpu.VMEM((_BA, _N_PAD), jnp.float32)],
        compiler_params=pltpu.CompilerParams(
            dimension_semantics=("arbitrary", "arbitrary")),
    )(ae, ae, cand_cat)


def kernel(out1, out2, anchor1, anchor2):
    ae1, ae2 = _gather(anchor1.astype(jnp.int32), anchor2.astype(jnp.int32),
                       out1, out2)
    ae = jnp.concatenate([ae1, ae2], axis=0)
    npad = _N_PAD - _N_NODES
    pad = jnp.full((npad, _D_FEAT), 1e9, jnp.float32)
    cand_cat = jnp.concatenate([out2, pad, out1, pad], axis=0)
    loss = _main(ae, cand_cat)
    return loss[0, 0] / (_N_ANCHORS * _K)
